# baseline (speedup 1.0000x reference)
import jax
import jax.numpy as jnp
from jax.experimental import pallas as pl

HID = 64
NB = 512


def _lin(p, x):
    return x @ p["W"] + p["b"]


def _silu(x):
    return x * jax.nn.sigmoid(x)


def _ln(x, g, b):
    m = jnp.mean(x, axis=-1, keepdims=True)
    v = jnp.var(x, axis=-1, keepdims=True)
    return (x - m) / jnp.sqrt(v + 1e-5) * g + b


def _rbf(d, vmin, vmax, bins):
    centers = jnp.linspace(vmin, vmax, bins)
    ls = centers[1] - centers[0]
    gamma = 1.0 / ls ** 2
    return jnp.exp(-gamma * (d[:, None] - centers) ** 2)


def _egc(p, x, edge_index, edge_attr):
    src = edge_index[0]
    dst = edge_index[1]
    x_j = jnp.take(x, src, axis=0)
    x_i = jnp.take(x, dst, axis=0)
    gate = _lin(p["src_gate"], x_i) + _lin(p["dst_gate"], x_j) + _lin(p["edge_gate"], edge_attr)
    gate = _silu(gate)
    msg = gate * _lin(p["dst_update"], x_j)
    aggr = jax.ops.segment_sum(msg, dst, num_segments=x.shape[0])
    out = _lin(p["src_update"], x) + aggr
    out = _ln(out, p["ln_g"], p["ln_b"])
    out = _silu(out)
    out = out + x
    return out


def _head_kernel(pool_ref, w1_ref, b1_ref, w2_ref, b2_ref, out_ref):
    h = pool_ref[...] @ w1_ref[...] + b1_ref[...]
    h = h * jax.nn.sigmoid(h)
    o = h @ w2_ref[...] + b2_ref[...]
    out_ref[...] = o[:, 0]


def kernel(g_x, g_edge_index, g_edge_attr, lg_edge_index, lg_edge_attr, batch, params):
    x = _silu(_ln(_lin(params["atom_emb"], g_x), params["atom_emb"]["ln_g"], params["atom_emb"]["ln_b"]))
    e = _rbf(g_edge_attr, 0.0, 10.0, 40)
    e = _silu(_ln(_lin(params["edge_emb"], e), params["edge_emb"]["ln_g"], params["edge_emb"]["ln_b"]))
    a = _rbf(lg_edge_attr, -1.0, 1.0, 20)
    a = _silu(_ln(_lin(params["angle_emb"], a), params["angle_emb"]["ln_g"], params["angle_emb"]["ln_b"]))
    for lp in params["alignn"]:
        e = _egc(lp["edge"], e, lg_edge_index, a)
        x = _egc(lp["node"], x, g_edge_index, e)
    for gp in params["gcn"]:
        x = _egc(gp, x, g_edge_index, e)
    ones = jnp.ones((x.shape[0], 1), jnp.float32)
    sums = jax.ops.segment_sum(x, batch, num_segments=NB)
    cnt = jax.ops.segment_sum(ones, batch, num_segments=NB)
    pool = sums / jnp.maximum(cnt, 1.0)

    out = pl.pallas_call(
        _head_kernel,
        out_shape=jax.ShapeDtypeStruct((NB,), jnp.float32),
    )(pool, params["conv_to_fc"]["W"], params["conv_to_fc"]["b"],
      params["out"]["W"], params["out"]["b"])
    return out


# speedup vs baseline: 1.2063x; 1.2063x over previous
import functools

import jax
import jax.numpy as jnp
from jax import lax
from jax.experimental import pallas as pl
from jax.experimental.pallas import tpu as pltpu
from jax.experimental.pallas import tpu_sc as plsc

HID = 64
NB = 512
N = 50000
E = 800000
ELG = 800000

CN = 512
BE = 256
NP_ = 50176
EP_ = 800256
NCH_N = NP_ // CN
NCH_E = EP_ // CN
CSP_N = 104
CSP_E = 1568

NTILES = 32

_mesh = plsc.VectorSubcoreMesh(core_axis_name="c", subcore_axis_name="s")


def _silu(x):
    return x * jax.nn.sigmoid(x)


def _ln(x, g, b):
    m = jnp.mean(x, axis=-1, keepdims=True)
    v = jnp.var(x, axis=-1, keepdims=True)
    return (x - m) / jnp.sqrt(v + 1e-5) * g + b


def _embx_body(x_ref, w_ref, b_ref, g_ref, bb_ref, o_ref):
    h = jnp.dot(x_ref[...], w_ref[...], preferred_element_type=jnp.float32)
    h = h + b_ref[...]
    h = _ln(h, g_ref[...], bb_ref[...])
    o_ref[...] = _silu(h)


def _emb_x(g_x_pad, p):
    blk = 1024
    grid = NP_ // blk
    return pl.pallas_call(
        _embx_body,
        grid=(grid,),
        in_specs=[
            pl.BlockSpec((blk, 128), lambda i: (i, 0)),
            pl.BlockSpec((128, HID), lambda i: (0, 0)),
            pl.BlockSpec((1, HID), lambda i: (0, 0)),
            pl.BlockSpec((1, HID), lambda i: (0, 0)),
            pl.BlockSpec((1, HID), lambda i: (0, 0)),
        ],
        out_specs=pl.BlockSpec((blk, HID), lambda i: (i, 0)),
        out_shape=jax.ShapeDtypeStruct((NP_, HID), jnp.float32),
    )(g_x_pad, p["W"], p["b"].reshape(1, HID), p["ln_g"].reshape(1, HID),
      p["ln_b"].reshape(1, HID))


def _rbf_body(bins, vmin, vmax, d_ref, w_ref, b_ref, g_ref, bb_ref, o_ref):
    d = d_ref[...]
    centers = vmin + lax.broadcasted_iota(jnp.int32, (1, bins), 1).astype(
        jnp.float32) * ((vmax - vmin) / (bins - 1))
    ls = (vmax - vmin) / (bins - 1)
    gamma = 1.0 / (ls * ls)
    z = jnp.exp(-gamma * (d - centers) ** 2)
    h = jnp.dot(z, w_ref[...], preferred_element_type=jnp.float32) + b_ref[...]
    h = _ln(h, g_ref[...], bb_ref[...])
    o_ref[...] = _silu(h)


def _emb_rbf(attr_pad, p, bins, vmin, vmax):
    blk = 1536
    grid = EP_ // blk
    return pl.pallas_call(
        functools.partial(_rbf_body, bins, vmin, vmax),
        grid=(grid,),
        in_specs=[
            pl.BlockSpec((blk, 1), lambda i: (i, 0)),
            pl.BlockSpec((bins, HID), lambda i: (0, 0)),
            pl.BlockSpec((1, HID), lambda i: (0, 0)),
            pl.BlockSpec((1, HID), lambda i: (0, 0)),
            pl.BlockSpec((1, HID), lambda i: (0, 0)),
        ],
        out_specs=pl.BlockSpec((blk, HID), lambda i: (i, 0)),
        out_shape=jax.ShapeDtypeStruct((EP_, HID), jnp.float32),
    )(attr_pad.reshape(EP_, 1), p["W"], p["b"].reshape(1, HID),
      p["ln_g"].reshape(1, HID), p["ln_b"].reshape(1, HID))


def _prep_body(x_ref, wsg_ref, wdg_ref, wdu_ref, bdu_ref, wsu_ref, bsu_ref,
               p_ref, qu_ref, s_ref):
    x = x_ref[...]
    p_ref[...] = jnp.dot(x, wsg_ref[...], preferred_element_type=jnp.float32)
    q = jnp.dot(x, wdg_ref[...], preferred_element_type=jnp.float32)
    u = jnp.dot(x, wdu_ref[...], preferred_element_type=jnp.float32) + bdu_ref[...]
    qu_ref[...] = jnp.concatenate([q, u], axis=-1)
    s_ref[...] = jnp.dot(x, wsu_ref[...], preferred_element_type=jnp.float32) + bsu_ref[...]


def _prep(xa, p, mp, blk):
    grid = mp // blk
    w = pl.BlockSpec((HID, HID), lambda i: (0, 0))
    b = pl.BlockSpec((1, HID), lambda i: (0, 0))
    return pl.pallas_call(
        _prep_body,
        grid=(grid,),
        in_specs=[pl.BlockSpec((blk, HID), lambda i: (i, 0)), w, w, w, b, w, b],
        out_specs=[
            pl.BlockSpec((blk, HID), lambda i: (i, 0)),
            pl.BlockSpec((blk, 2 * HID), lambda i: (i, 0)),
            pl.BlockSpec((blk, HID), lambda i: (i, 0)),
        ],
        out_shape=[
            jax.ShapeDtypeStruct((mp, HID), jnp.float32),
            jax.ShapeDtypeStruct((mp, 2 * HID), jnp.float32),
            jax.ShapeDtypeStruct((mp, HID), jnp.float32),
        ],
    )(xa, p["src_gate"]["W"], p["dst_gate"]["W"], p["dst_update"]["W"],
      p["dst_update"]["b"].reshape(1, HID), p["src_update"]["W"],
      p["src_update"]["b"].reshape(1, HID))


def _egprep_body(e_ref, w_ref, b_ref, o_ref):
    o_ref[...] = jnp.dot(e_ref[...], w_ref[...],
                         preferred_element_type=jnp.float32) + b_ref[...]


def _egprep(eat, p):
    btot = (p["src_gate"]["b"] + p["dst_gate"]["b"] + p["edge_gate"]["b"]).reshape(1, HID)
    blk = 2000
    grid = E // blk
    return pl.pallas_call(
        _egprep_body,
        grid=(grid,),
        in_specs=[
            pl.BlockSpec((blk, HID), lambda i: (i, 0)),
            pl.BlockSpec((HID, HID), lambda i: (0, 0)),
            pl.BlockSpec((1, HID), lambda i: (0, 0)),
        ],
        out_specs=pl.BlockSpec((blk, HID), lambda i: (i, 0)),
        out_shape=jax.ShapeDtypeStruct((E, HID), jnp.float32),
    )(eat, p["edge_gate"]["W"], btot)


def _combine_body(x_ref, s_ref, a_ref, g_ref, b_ref, o_ref):
    y = s_ref[...] + a_ref[...]
    y = _ln(y, g_ref[...], b_ref[...])
    o_ref[...] = _silu(y) + x_ref[...]


def _combine(xa, s, aggr, p, mp, blk):
    grid = mp // blk
    return pl.pallas_call(
        _combine_body,
        grid=(grid,),
        in_specs=[
            pl.BlockSpec((blk, HID), lambda i: (i, 0)),
            pl.BlockSpec((blk, HID), lambda i: (i, 0)),
            pl.BlockSpec((blk, HID), lambda i: (i, 0)),
            pl.BlockSpec((1, HID), lambda i: (0, 0)),
            pl.BlockSpec((1, HID), lambda i: (0, 0)),
        ],
        out_specs=pl.BlockSpec((blk, HID), lambda i: (i, 0)),
        out_shape=jax.ShapeDtypeStruct((mp, HID), jnp.float32),
    )(xa, s, aggr, p["ln_g"].reshape(1, HID), p["ln_b"].reshape(1, HID))


def _head_body(part_ref, cnt_ref, w1_ref, b1_ref, w2_ref, b2_ref, out_ref):
    sums = jnp.sum(part_ref[...], axis=0)
    pool = sums / cnt_ref[...]
    h = jnp.dot(pool, w1_ref[...], preferred_element_type=jnp.float32) + b1_ref[...]
    h = _silu(h)
    o = jnp.dot(h, w2_ref[...], preferred_element_type=jnp.float32) + b2_ref[...]
    out_ref[...] = o[:, 0]


def _head(partials, cnt, params):
    return pl.pallas_call(
        _head_body,
        out_shape=jax.ShapeDtypeStruct((NB,), jnp.float32),
    )(partials, cnt, params["conv_to_fc"]["W"],
      params["conv_to_fc"]["b"].reshape(1, HID),
      params["out"]["W"], params["out"]["b"].reshape(1, 1))


def _sld(ref, idx):
    return ref[pl.ds(idx, 16)][0]


def _make_sc_aggregate(mp, nchunk, csp):
    nt = (nchunk + NTILES - 1) // NTILES

    @functools.partial(
        pl.kernel,
        out_type=jax.ShapeDtypeStruct((mp * HID,), jnp.float32),
        mesh=_mesh,
        scratch_types=[
            pltpu.VMEM((csp + 16,), jnp.int32),
            pltpu.VMEM((CN * HID,), jnp.float32),
            pltpu.VMEM((CN * HID,), jnp.float32),
            pltpu.VMEM((BE * HID,), jnp.float32),
            pltpu.VMEM((BE, 2 * HID), jnp.float32),
            pltpu.VMEM((BE + 16,), jnp.int32),
            pltpu.VMEM((BE,), jnp.int32),
        ],
    )
    def k(p_hbm, qu_hbm, eg_hbm, sdst_hbm, src_hbm, cs_hbm, aggr_hbm,
          csbuf, pbuf, obuf, egbuf, qubuf, dbuf, ibuf):
        wid = lax.axis_index("c") * 16 + lax.axis_index("s")
        pltpu.sync_copy(cs_hbm, csbuf.at[pl.ds(0, csp)])

        def chunk_body(t, _):
            chunk = wid + NTILES * t

            @pl.when(chunk < nchunk)
            def _():
                node_base = chunk * CN
                e0 = _sld(csbuf, chunk)
                e1 = _sld(csbuf, chunk + 1)
                pltpu.sync_copy(p_hbm.at[pl.ds(node_base * HID, CN * HID)], pbuf)

                @pl.loop(0, CN * HID, step=16)
                def _zero(i):
                    obuf[pl.ds(i, 16)] = jnp.zeros((16,), jnp.float32)

                eb0 = (e0 // BE) * BE
                nblk = (e1 - eb0 + BE - 1) // BE

                def blk_body(b, _):
                    eb = eb0 + b * BE
                    pltpu.sync_copy(eg_hbm.at[pl.ds(eb * HID, BE * HID)], egbuf)
                    pltpu.sync_copy(sdst_hbm.at[pl.ds(eb, BE)], dbuf.at[pl.ds(0, BE)])
                    pltpu.sync_copy(src_hbm.at[pl.ds(eb, BE)], ibuf)
                    pltpu.sync_copy(qu_hbm.at[ibuf.at[pl.ds(0, 128)]],
                                    qubuf.at[pl.ds(0, 128)])
                    pltpu.sync_copy(qu_hbm.at[ibuf.at[pl.ds(128, 128)]],
                                    qubuf.at[pl.ds(128, 128)])

                    def grp_body(g, _):
                        goff = pl.multiple_of(g * 8, 8)
                        qu8 = qubuf.at[pl.ds(goff, 8)]
                        for kk in range(8):
                            rl = g * 8 + kk
                            r = eb + rl
                            ok = jnp.logical_and(r >= e0, r < e1)
                            scale = jnp.where(ok, 1.0, 0.0).astype(jnp.float32)
                            s = jnp.clip(_sld(dbuf, rl) - node_base, 0, CN - 1)
                            for c in range(4):
                                sl16 = pl.ds(c * 16, 16)
                                z = (pbuf[pl.ds(s * HID + c * 16, 16)]
                                     + qu8[kk, sl16]
                                     + egbuf[pl.ds(rl * HID + c * 16, 16)])
                                gate = z / (1.0 + jnp.exp(-z))
                                m = gate * qu8[kk, pl.ds(HID + c * 16, 16)] * scale
                                plsc.addupdate(
                                    obuf.at[pl.ds(s * HID + c * 16, 16)], m)
                        return 0

                    lax.fori_loop(0, BE // 8, grp_body, 0)
                    return 0

                lax.fori_loop(0, nblk, blk_body, 0)
                pltpu.sync_copy(obuf, aggr_hbm.at[pl.ds(node_base * HID, CN * HID)])

            return 0

        lax.fori_loop(0, nt, chunk_body, 0)

    return k


_sc_aggregate_node = _make_sc_aggregate(NP_, NCH_N, CSP_N)
_sc_aggregate_edge = _make_sc_aggregate(EP_, NCH_E, CSP_E)


def _make_sc_pool():
    span = 1568

    @functools.partial(
        pl.kernel,
        out_type=jax.ShapeDtypeStruct((NTILES, NB * HID), jnp.float32),
        mesh=_mesh,
        scratch_types=[
            pltpu.VMEM((NB * HID,), jnp.float32),
            pltpu.VMEM((BE * HID,), jnp.float32),
            pltpu.VMEM((BE + 16,), jnp.int32),
        ],
    )
    def k(x_hbm, batch_hbm, part_hbm, obuf, xbuf, bbuf):
        wid = lax.axis_index("c") * 16 + lax.axis_index("s")
        r0 = wid * span
        r_hi = jnp.minimum(N, r0 + span)

        @pl.loop(0, NB * HID, step=16)
        def _zero(i):
            obuf[pl.ds(i, 16)] = jnp.zeros((16,), jnp.float32)

        nblk = (r_hi - r0 + BE - 1) // BE

        def blk_body(b, _):
            rb = r0 + b * BE
            pltpu.sync_copy(x_hbm.at[pl.ds(rb * HID, BE * HID)], xbuf)
            pltpu.sync_copy(batch_hbm.at[pl.ds(rb, BE)], bbuf.at[pl.ds(0, BE)])

            def row_body(rl, _):
                r = rb + rl
                ok = r < r_hi
                scale = jnp.where(ok, 1.0, 0.0).astype(jnp.float32)
                s = jnp.clip(_sld(bbuf, rl), 0, NB - 1)
                for c in range(4):
                    v = xbuf[pl.ds(rl * HID + c * 16, 16)] * scale
                    plsc.addupdate(obuf.at[pl.ds(s * HID + c * 16, 16)], v)
                return 0

            lax.fori_loop(0, BE, row_body, 0)
            return 0

        lax.fori_loop(0, nblk, blk_body, 0)
        pltpu.sync_copy(obuf, part_hbm.at[wid])

    return k


_sc_pool = _make_sc_pool()


def _egc(p, xa, src2d, sdst, cs, eat, mp, blk, sc_agg):
    P, QU, S = _prep(xa, p, mp, blk)
    Eg = _egprep(eat, p)
    aggr = sc_agg(P.reshape(-1), QU, Eg.reshape(-1), sdst, src2d, cs)
    return _combine(xa, S, aggr.reshape(mp, HID), p, mp, blk)


def kernel(g_x, g_edge_index, g_edge_attr, lg_edge_index, lg_edge_attr, batch, params):
    i32 = jnp.int32
    gsrc, gdst = g_edge_index[0], g_edge_index[1]
    perm_g = jnp.argsort(gdst)
    sg_src = gsrc[perm_g]
    sg_dst = gdst[perm_g]
    eat_g = g_edge_attr[perm_g]
    inv_g = jnp.zeros((E,), i32).at[perm_g].set(jnp.arange(E, dtype=i32))
    lsrc = inv_g[lg_edge_index[0]]
    ldst = inv_g[lg_edge_index[1]]
    perm_l = jnp.argsort(ldst)
    sl_src = lsrc[perm_l]
    sl_dst = ldst[perm_l]
    lat = lg_edge_attr[perm_l]

    bnd_n = jnp.arange(NCH_N + 1, dtype=i32) * CN
    cs_g = jnp.searchsorted(sg_dst, bnd_n, side="left").astype(i32)
    cs_g = jnp.concatenate([cs_g, jnp.full((CSP_N - NCH_N - 1,), E, i32)])
    bnd_e = jnp.arange(NCH_E + 1, dtype=i32) * CN
    cs_l = jnp.searchsorted(sl_dst, bnd_e, side="left").astype(i32)
    cs_l = jnp.concatenate([cs_l, jnp.full((CSP_E - NCH_E - 1,), ELG, i32)])

    offb = jnp.searchsorted(batch, jnp.arange(NB + 1, dtype=i32), side="left")
    cnt = jnp.maximum((offb[1:] - offb[:-1]).astype(jnp.float32), 1.0).reshape(NB, 1)

    g_x_pad = jnp.pad(g_x, ((0, NP_ - N), (0, 0)))
    eat_g_pad = jnp.pad(eat_g, (0, EP_ - E))
    lat_pad = jnp.pad(lat, (0, EP_ - ELG))

    x = _emb_x(g_x_pad, params["atom_emb"])
    e = _emb_rbf(eat_g_pad, params["edge_emb"], 40, 0.0, 10.0)
    a = _emb_rbf(lat_pad, params["angle_emb"], 20, -1.0, 1.0)

    for lp in params["alignn"]:
        e = _egc(lp["edge"], e, sl_src, sl_dst, cs_l, a, EP_, 1536,
                 _sc_aggregate_edge)
        x = _egc(lp["node"], x, sg_src, sg_dst, cs_g, e, NP_, 1024,
                 _sc_aggregate_node)
    for gp in params["gcn"]:
        x = _egc(gp, x, sg_src, sg_dst, cs_g, e, NP_, 1024,
                 _sc_aggregate_node)

    partials = _sc_pool(x.reshape(-1), batch).reshape(NTILES, NB, HID)
    return _head(partials, cnt, params)
